# baseline (speedup 1.0000x reference)
import functools

import jax
import jax.numpy as jnp
from jax import lax
from jax.experimental import pallas as pl
from jax.experimental.pallas import tpu as pltpu
from jax.experimental.pallas import tpu_sc as plsc

B = 8
C = 19
N = 512 * 512
K = max(int(N * 0.2), 1)

NB = 1024
SHIFT1 = 22
SHIFT2 = 12

NW = 32
BH = 4
SH = NW // BH
PER_W = (BH * N) // NW
CHUNK = 4096
N_CHUNKS = PER_W // CHUNK
UNROLL = 4

RB = 256

def _sign():
    return jnp.uint32(0x80000000)


def _mant():
    return jnp.uint32(0x7FFFFFFF)


def _ce_key_body(logits_ref, tgt_ref, out_ref):
    x = logits_ref[0]
    t = tgt_ref[0]
    cls = lax.broadcasted_iota(jnp.int32, (C, RB, 512), 0)
    xt = jnp.sum(jnp.where(cls == t[None], x, 0.0), axis=0)
    m = jnp.max(x, axis=0)
    s = jnp.sum(jnp.exp(x - m[None]), axis=0)
    ce = jnp.log(s) + m - xt
    bits = lax.bitcast_convert_type(ce, jnp.uint32)
    key = jnp.where(bits >= _sign(), ~bits, bits | _sign())
    out_ref[0] = key


def _ce_keys(logits, target_long, off):
    out = pl.pallas_call(
        _ce_key_body,
        grid=(BH, 512 // RB),
        in_specs=[
            pl.BlockSpec((1, C, RB, 512), lambda b, j: (b + off, 0, j, 0)),
            pl.BlockSpec((1, RB, 512), lambda b, j: (b + off, j, 0)),
        ],
        out_specs=pl.BlockSpec((1, RB, 512), lambda b, j: (b, j, 0)),
        out_shape=jax.ShapeDtypeStruct((BH, 512, 512), jnp.uint32),
    )(logits, target_long)
    return out.reshape(BH * N)


def _make_hist_kernel(shift, filtered):
    mesh = plsc.VectorSubcoreMesh(core_axis_name="c", subcore_axis_name="s")

    @functools.partial(
        pl.kernel,
        out_type=(
            jax.ShapeDtypeStruct((BH, 16), jnp.int32),
            jax.ShapeDtypeStruct((BH, 16), jnp.int32),
            jax.ShapeDtypeStruct((BH, 16), jnp.float32),
        ),
        mesh=mesh,
        compiler_params=pltpu.CompilerParams(needs_layout_passes=False),
        scratch_types=[
            pltpu.VMEM((NB * 16,), jnp.int32),
            pltpu.VMEM((NB * 16,), jnp.float32),
            pltpu.VMEM((NB,), jnp.int32),
            pltpu.VMEM((NB,), jnp.float32),
            pltpu.VMEM((NB,), jnp.int32),
            pltpu.VMEM((NB,), jnp.float32),
            pltpu.VMEM((CHUNK,), jnp.uint32),
            pltpu.VMEM((CHUNK,), jnp.uint32),
            pltpu.VMEM((32,), jnp.int32),
            pltpu.VMEM((16,), jnp.int32),
            pltpu.VMEM((16,), jnp.int32),
            pltpu.VMEM((16,), jnp.float32),
            pltpu.VMEM_SHARED((16, NB), jnp.int32),
            pltpu.VMEM_SHARED((16, NB), jnp.float32),
            pltpu.SemaphoreType.DMA,
            pltpu.SemaphoreType.DMA,
        ],
    )
    def hist(keys_hbm, filt_hbm, p_out, ca_out, sa_out,
             cnt_v, sum_v, cnt_f, sum_f, tmp_c, tmp_s, buf0, buf1, filt_v,
             ob_p, ob_c, ob_s, shared_c, shared_s, sem0, sem1):
        core = lax.axis_index("c")
        sidx = lax.axis_index("s")
        sample = core * (BH // 2) + sidx // SH
        shard = sidx % SH
        base = sample * N + shard * PER_W

        zi = jnp.zeros((16,), jnp.int32)
        zf = jnp.zeros((16,), jnp.float32)

        def zbody(i, carry):
            for u in range(4):
                off = (i * 4 + u) * 16
                cnt_v[pl.ds(off, 16)] = zi
                sum_v[pl.ds(off, 16)] = zf
            return carry

        lax.fori_loop(0, (NB * 16) // 64, zbody, 0)

        pltpu.sync_copy(filt_hbm, filt_v)
        if filtered:
            pfx = plsc.load_gather(filt_v, [jnp.full((16,), sample, jnp.int32)])
            pfx = pfx.astype(jnp.uint32)
            krem = plsc.load_gather(
                filt_v, [jnp.full((16,), 16 + sample, jnp.int32)])
        else:
            krem = jnp.full((16,), K, jnp.int32)

        lane = lax.broadcasted_iota(jnp.int32, (16,), 0)
        ones = jnp.ones((16,), jnp.int32)

        def make_inner(buf):
            def inner(g, carry):
                kvs = [buf[pl.ds((g * UNROLL + u) * 16, 16)]
                       for u in range(UNROLL)]
                idxs = []
                vals = []
                msks = []
                for kv in kvs:
                    bin_ = ((kv >> shift) & jnp.uint32(NB - 1)).astype(jnp.int32)
                    idxs.append((bin_ << 4) | lane)
                    bits = jnp.where(kv >= _sign(), kv & _mant(), ~kv)
                    vals.append(lax.bitcast_convert_type(bits, jnp.float32))
                    if filtered:
                        msks.append((kv >> SHIFT1) == pfx)
                for u in range(UNROLL):
                    if filtered:
                        plsc.addupdate_scatter(cnt_v, [idxs[u]], ones,
                                               mask=msks[u])
                        plsc.addupdate_scatter(sum_v, [idxs[u]], vals[u],
                                               mask=msks[u])
                    else:
                        plsc.addupdate_scatter(cnt_v, [idxs[u]], ones)
                        plsc.addupdate_scatter(sum_v, [idxs[u]], vals[u])
                return carry
            return inner

        bufs = (buf0, buf1)
        sems = (sem0, sem1)
        copies = [None, None]
        copies[0] = pltpu.async_copy(
            keys_hbm.at[pl.ds(base, CHUNK)], buf0, sem0)
        for ci in range(N_CHUNKS):
            cur = ci % 2
            nxt = (ci + 1) % 2
            if ci + 1 < N_CHUNKS:
                copies[nxt] = pltpu.async_copy(
                    keys_hbm.at[pl.ds(base + (ci + 1) * CHUNK, CHUNK)],
                    bufs[nxt], sems[nxt])
            copies[cur].wait()
            lax.fori_loop(0, CHUNK // (16 * UNROLL), make_inner(bufs[cur]), 0)

        def fold(g, carry):
            base16 = g * 256
            gidx = base16 + lane * 16
            ci_acc = plsc.load_gather(cnt_v, [gidx])
            cf_acc = plsc.load_gather(sum_v, [gidx])
            for l in range(1, 16):
                ci_acc = ci_acc + plsc.load_gather(cnt_v, [gidx + l])
                cf_acc = cf_acc + plsc.load_gather(sum_v, [gidx + l])
            cnt_f[pl.ds(g * 16, 16)] = ci_acc
            sum_f[pl.ds(g * 16, 16)] = cf_acc
            return carry

        lax.fori_loop(0, NB // 16, fold, 0)

        pltpu.sync_copy(cnt_f, shared_c.at[sidx])
        pltpu.sync_copy(sum_f, shared_s.at[sidx])
        plsc.subcore_barrier()

        @pl.when(shard == 0)
        def _select():
            for q in range(1, SH):
                pltpu.sync_copy(shared_c.at[sidx + q], tmp_c)
                pltpu.sync_copy(shared_s.at[sidx + q], tmp_s)

                def mbody(g, carry):
                    off = g * 16
                    cnt_f[pl.ds(off, 16)] = (cnt_f[pl.ds(off, 16)]
                                             + tmp_c[pl.ds(off, 16)])
                    sum_f[pl.ds(off, 16)] = (sum_f[pl.ds(off, 16)]
                                             + tmp_s[pl.ds(off, 16)])
                    return carry

                lax.fori_loop(0, NB // 16, mbody, 0)

            def tbody(g, carry):
                ai, af = carry
                return (ai + cnt_f[pl.ds(g * 16, 16)],
                        af + sum_f[pl.ds(g * 16, 16)])

            ti, tf = lax.fori_loop(
                0, NB // 16, tbody,
                (jnp.zeros((16,), jnp.int32), jnp.zeros((16,), jnp.float32)))
            tot_c = jnp.sum(ti)
            tot_s = jnp.sum(tf)

            def sbody(g, carry):
                run_c, run_s, accp, accc, accs = carry
                v = cnt_f[pl.ds(g * 16, 16)]
                w = sum_f[pl.ds(g * 16, 16)]
                incl_c = run_c + plsc.cumsum(v)
                incl_s = run_s + plsc.cumsum(w)
                gt_c = tot_c - incl_c
                gt_s = tot_s - incl_s
                flag = (gt_c < krem) & ((gt_c + v) >= krem)
                accp = accp + jnp.where(flag, g * 16 + lane, 0)
                accc = accc + jnp.where(flag, gt_c, 0)
                accs = accs + jnp.where(flag, gt_s, jnp.float32(0.0))
                return (run_c + jnp.sum(v), run_s + jnp.sum(w),
                        accp, accc, accs)

            z16i = jnp.zeros((16,), jnp.int32)
            _, _, accp, accc, accs = lax.fori_loop(
                0, NB // 16, sbody,
                (jnp.int32(0), jnp.float32(0.0), z16i, z16i,
                 jnp.zeros((16,), jnp.float32)))
            ob_p[...] = accp
            ob_c[...] = accc
            ob_s[...] = accs
            pltpu.sync_copy(ob_p, p_out.at[sample])
            pltpu.sync_copy(ob_c, ca_out.at[sample])
            pltpu.sync_copy(ob_s, sa_out.at[sample])

    return hist


@functools.lru_cache(maxsize=None)
def _hist_kernels():
    return (_make_hist_kernel(SHIFT1, filtered=False),
            _make_hist_kernel(SHIFT2, filtered=True))


def _decode_key(key_u32):
    neg = key_u32 < _sign()
    bits = jnp.where(neg, ~key_u32, key_u32 & _mant())
    return lax.bitcast_convert_type(bits, jnp.float32)


def _half(logits, target_long, off):
    _hist_pass1, _hist_pass2 = _hist_kernels()
    keys = _ce_keys(logits, target_long, off)

    p1v, ca1v, sa1v = _hist_pass1(keys, jnp.zeros((32,), jnp.int32))
    P1 = p1v.sum(axis=1)
    cA1 = ca1v.sum(axis=1)
    sA1 = sa1v.sum(axis=1)
    r1 = jnp.full((BH,), K, jnp.int32) - cA1

    filt = jnp.concatenate([
        jnp.zeros((16,), jnp.int32).at[:BH].set(P1),
        jnp.zeros((16,), jnp.int32).at[:BH].set(r1),
    ])
    p2v, ca2v, sa2v = _hist_pass2(keys, filt)
    P2 = p2v.sum(axis=1)
    cA2 = ca2v.sum(axis=1)
    sA2 = sa2v.sum(axis=1)

    r = (r1 - cA2).astype(jnp.float32)
    key_mid = ((P1.astype(jnp.uint32) << SHIFT1)
               | (P2.astype(jnp.uint32) << SHIFT2)
               | jnp.uint32(1 << (SHIFT2 - 1)))
    t_mid = _decode_key(key_mid)
    return (sA1 + sA2 + r * t_mid) / jnp.float32(K)


def kernel(logits, target_long):
    per_sample = jnp.concatenate([
        _half(logits, target_long, 0),
        _half(logits, target_long, BH),
    ])
    return jnp.mean(per_sample)


# speedup vs baseline: 1.1362x; 1.1362x over previous
import functools

import jax
import jax.numpy as jnp
from jax import lax
from jax.experimental import pallas as pl
from jax.experimental.pallas import tpu as pltpu
from jax.experimental.pallas import tpu_sc as plsc

B = 8
C = 19
N = 512 * 512
K = max(int(N * 0.2), 1)

NB = 1024
SHIFT1 = 22
SHIFT2 = 12

NW = 32
BH = 8
SH = NW // BH
PER_W = (BH * N) // NW
CHUNK = 4096
N_CHUNKS = PER_W // CHUNK
UNROLL = 4

RB = 256

def _sign():
    return jnp.uint32(0x80000000)


def _mant():
    return jnp.uint32(0x7FFFFFFF)


def _ce_key_body(logits_ref, tgt_ref, out_ref):
    x = logits_ref[0]
    t = tgt_ref[0]
    cls = lax.broadcasted_iota(jnp.int32, (C, RB, 512), 0)
    xt = jnp.sum(jnp.where(cls == t[None], x, 0.0), axis=0)
    m = jnp.max(x, axis=0)
    s = jnp.sum(jnp.exp(x - m[None]), axis=0)
    ce = jnp.log(s) + m - xt
    bits = lax.bitcast_convert_type(ce, jnp.uint32)
    key = jnp.where(bits >= _sign(), ~bits, bits | _sign())
    out_ref[0] = key


def _ce_keys(logits, target_long, off):
    out = pl.pallas_call(
        _ce_key_body,
        grid=(BH, 512 // RB),
        in_specs=[
            pl.BlockSpec((1, C, RB, 512), lambda b, j: (b + off, 0, j, 0)),
            pl.BlockSpec((1, RB, 512), lambda b, j: (b + off, j, 0)),
        ],
        out_specs=pl.BlockSpec((1, RB, 512), lambda b, j: (b, j, 0)),
        out_shape=jax.ShapeDtypeStruct((BH, 512, 512), jnp.uint32),
    )(logits, target_long)
    return out.reshape(BH * N)


def _make_hist_kernel(shift, filtered):
    mesh = plsc.VectorSubcoreMesh(core_axis_name="c", subcore_axis_name="s")

    @functools.partial(
        pl.kernel,
        out_type=(
            jax.ShapeDtypeStruct((BH, 16), jnp.int32),
            jax.ShapeDtypeStruct((BH, 16), jnp.int32),
            jax.ShapeDtypeStruct((BH, 16), jnp.float32),
        ),
        mesh=mesh,
        compiler_params=pltpu.CompilerParams(needs_layout_passes=False),
        scratch_types=[
            pltpu.VMEM((NB * 16,), jnp.int32),
            pltpu.VMEM((NB * 16,), jnp.float32),
            pltpu.VMEM((NB,), jnp.int32),
            pltpu.VMEM((NB,), jnp.float32),
            pltpu.VMEM((NB,), jnp.int32),
            pltpu.VMEM((NB,), jnp.float32),
            pltpu.VMEM((CHUNK,), jnp.uint32),
            pltpu.VMEM((CHUNK,), jnp.uint32),
            pltpu.VMEM((32,), jnp.int32),
            pltpu.VMEM((16,), jnp.int32),
            pltpu.VMEM((16,), jnp.int32),
            pltpu.VMEM((16,), jnp.float32),
            pltpu.VMEM_SHARED((16, NB), jnp.int32),
            pltpu.VMEM_SHARED((16, NB), jnp.float32),
            pltpu.SemaphoreType.DMA,
            pltpu.SemaphoreType.DMA,
        ],
    )
    def hist(keys_hbm, filt_hbm, p_out, ca_out, sa_out,
             cnt_v, sum_v, cnt_f, sum_f, tmp_c, tmp_s, buf0, buf1, filt_v,
             ob_p, ob_c, ob_s, shared_c, shared_s, sem0, sem1):
        core = lax.axis_index("c")
        sidx = lax.axis_index("s")
        sample = core * (BH // 2) + sidx // SH
        shard = sidx % SH
        base = sample * N + shard * PER_W

        zi = jnp.zeros((16,), jnp.int32)
        zf = jnp.zeros((16,), jnp.float32)

        def zbody(i, carry):
            for u in range(4):
                off = (i * 4 + u) * 16
                cnt_v[pl.ds(off, 16)] = zi
                sum_v[pl.ds(off, 16)] = zf
            return carry

        lax.fori_loop(0, (NB * 16) // 64, zbody, 0)

        pltpu.sync_copy(filt_hbm, filt_v)
        if filtered:
            pfx = plsc.load_gather(filt_v, [jnp.full((16,), sample, jnp.int32)])
            pfx = pfx.astype(jnp.uint32)
            krem = plsc.load_gather(
                filt_v, [jnp.full((16,), 16 + sample, jnp.int32)])
        else:
            krem = jnp.full((16,), K, jnp.int32)

        lane = lax.broadcasted_iota(jnp.int32, (16,), 0)
        ones = jnp.ones((16,), jnp.int32)

        def make_inner(buf):
            def inner(g, carry):
                kvs = [buf[pl.ds((g * UNROLL + u) * 16, 16)]
                       for u in range(UNROLL)]
                idxs = []
                vals = []
                msks = []
                for kv in kvs:
                    bin_ = ((kv >> shift) & jnp.uint32(NB - 1)).astype(jnp.int32)
                    idxs.append((bin_ << 4) | lane)
                    bits = jnp.where(kv >= _sign(), kv & _mant(), ~kv)
                    vals.append(lax.bitcast_convert_type(bits, jnp.float32))
                    if filtered:
                        msks.append((kv >> SHIFT1) == pfx)
                for u in range(UNROLL):
                    if filtered:
                        plsc.addupdate_scatter(cnt_v, [idxs[u]], ones,
                                               mask=msks[u])
                        plsc.addupdate_scatter(sum_v, [idxs[u]], vals[u],
                                               mask=msks[u])
                    else:
                        plsc.addupdate_scatter(cnt_v, [idxs[u]], ones)
                        plsc.addupdate_scatter(sum_v, [idxs[u]], vals[u])
                return carry
            return inner

        bufs = (buf0, buf1)
        sems = (sem0, sem1)
        copies = [None, None]
        copies[0] = pltpu.async_copy(
            keys_hbm.at[pl.ds(base, CHUNK)], buf0, sem0)
        for ci in range(N_CHUNKS):
            cur = ci % 2
            nxt = (ci + 1) % 2
            if ci + 1 < N_CHUNKS:
                copies[nxt] = pltpu.async_copy(
                    keys_hbm.at[pl.ds(base + (ci + 1) * CHUNK, CHUNK)],
                    bufs[nxt], sems[nxt])
            copies[cur].wait()
            lax.fori_loop(0, CHUNK // (16 * UNROLL), make_inner(bufs[cur]), 0)

        def fold(g, carry):
            base16 = g * 256
            gidx = base16 + lane * 16
            ci_acc = plsc.load_gather(cnt_v, [gidx])
            cf_acc = plsc.load_gather(sum_v, [gidx])
            for l in range(1, 16):
                ci_acc = ci_acc + plsc.load_gather(cnt_v, [gidx + l])
                cf_acc = cf_acc + plsc.load_gather(sum_v, [gidx + l])
            cnt_f[pl.ds(g * 16, 16)] = ci_acc
            sum_f[pl.ds(g * 16, 16)] = cf_acc
            return carry

        lax.fori_loop(0, NB // 16, fold, 0)

        pltpu.sync_copy(cnt_f, shared_c.at[sidx])
        pltpu.sync_copy(sum_f, shared_s.at[sidx])
        plsc.subcore_barrier()

        @pl.when(shard == 0)
        def _select():
            for q in range(1, SH):
                pltpu.sync_copy(shared_c.at[sidx + q], tmp_c)
                pltpu.sync_copy(shared_s.at[sidx + q], tmp_s)

                def mbody(g, carry):
                    off = g * 16
                    cnt_f[pl.ds(off, 16)] = (cnt_f[pl.ds(off, 16)]
                                             + tmp_c[pl.ds(off, 16)])
                    sum_f[pl.ds(off, 16)] = (sum_f[pl.ds(off, 16)]
                                             + tmp_s[pl.ds(off, 16)])
                    return carry

                lax.fori_loop(0, NB // 16, mbody, 0)

            def tbody(g, carry):
                ai, af = carry
                return (ai + cnt_f[pl.ds(g * 16, 16)],
                        af + sum_f[pl.ds(g * 16, 16)])

            ti, tf = lax.fori_loop(
                0, NB // 16, tbody,
                (jnp.zeros((16,), jnp.int32), jnp.zeros((16,), jnp.float32)))
            tot_c = jnp.sum(ti)
            tot_s = jnp.sum(tf)

            def sbody(g, carry):
                run_c, run_s, accp, accc, accs = carry
                v = cnt_f[pl.ds(g * 16, 16)]
                w = sum_f[pl.ds(g * 16, 16)]
                incl_c = run_c + plsc.cumsum(v)
                incl_s = run_s + plsc.cumsum(w)
                gt_c = tot_c - incl_c
                gt_s = tot_s - incl_s
                flag = (gt_c < krem) & ((gt_c + v) >= krem)
                accp = accp + jnp.where(flag, g * 16 + lane, 0)
                accc = accc + jnp.where(flag, gt_c, 0)
                accs = accs + jnp.where(flag, gt_s, jnp.float32(0.0))
                return (run_c + jnp.sum(v), run_s + jnp.sum(w),
                        accp, accc, accs)

            z16i = jnp.zeros((16,), jnp.int32)
            _, _, accp, accc, accs = lax.fori_loop(
                0, NB // 16, sbody,
                (jnp.int32(0), jnp.float32(0.0), z16i, z16i,
                 jnp.zeros((16,), jnp.float32)))
            ob_p[...] = accp
            ob_c[...] = accc
            ob_s[...] = accs
            pltpu.sync_copy(ob_p, p_out.at[sample])
            pltpu.sync_copy(ob_c, ca_out.at[sample])
            pltpu.sync_copy(ob_s, sa_out.at[sample])

    return hist


@functools.lru_cache(maxsize=None)
def _hist_kernels():
    return (_make_hist_kernel(SHIFT1, filtered=False),
            _make_hist_kernel(SHIFT2, filtered=True))


def _decode_key(key_u32):
    neg = key_u32 < _sign()
    bits = jnp.where(neg, ~key_u32, key_u32 & _mant())
    return lax.bitcast_convert_type(bits, jnp.float32)


def _half(logits, target_long, off):
    _hist_pass1, _hist_pass2 = _hist_kernels()
    keys = _ce_keys(logits, target_long, off)

    p1v, ca1v, sa1v = _hist_pass1(keys, jnp.zeros((32,), jnp.int32))
    P1 = p1v.sum(axis=1)
    cA1 = ca1v.sum(axis=1)
    sA1 = sa1v.sum(axis=1)
    r1 = jnp.full((BH,), K, jnp.int32) - cA1

    filt = jnp.concatenate([
        jnp.zeros((16,), jnp.int32).at[:BH].set(P1),
        jnp.zeros((16,), jnp.int32).at[:BH].set(r1),
    ])
    p2v, ca2v, sa2v = _hist_pass2(keys, filt)
    P2 = p2v.sum(axis=1)
    cA2 = ca2v.sum(axis=1)
    sA2 = sa2v.sum(axis=1)

    r = (r1 - cA2).astype(jnp.float32)
    key_mid = ((P1.astype(jnp.uint32) << SHIFT1)
               | (P2.astype(jnp.uint32) << SHIFT2)
               | jnp.uint32(1 << (SHIFT2 - 1)))
    t_mid = _decode_key(key_mid)
    return (sA1 + sA2 + r * t_mid) / jnp.float32(K)


def kernel(logits, target_long):
    return jnp.mean(_half(logits, target_long, 0))


# speedup vs baseline: 1.1919x; 1.0490x over previous
import functools

import jax
import jax.numpy as jnp
from jax import lax
from jax.experimental import pallas as pl
from jax.experimental.pallas import tpu as pltpu
from jax.experimental.pallas import tpu_sc as plsc

B = 8
C = 19
N = 512 * 512
K = max(int(N * 0.2), 1)

NB = 1024
SHIFT1 = 22
SHIFT2 = 12

NW = 32
BH = 8
SH = NW // BH
PER_W = (BH * N) // NW
CHUNK = 8192
N_CHUNKS = PER_W // CHUNK
UNROLL = 8

RB = 256

def _sign():
    return jnp.uint32(0x80000000)


def _mant():
    return jnp.uint32(0x7FFFFFFF)


def _ce_key_body(logits_ref, tgt_ref, out_ref):
    x = logits_ref[0]
    t = tgt_ref[0]
    cls = lax.broadcasted_iota(jnp.int32, (C, RB, 512), 0)
    xt = jnp.sum(jnp.where(cls == t[None], x, 0.0), axis=0)
    m = jnp.max(x, axis=0)
    s = jnp.sum(jnp.exp(x - m[None]), axis=0)
    ce = jnp.log(s) + m - xt
    bits = lax.bitcast_convert_type(ce, jnp.uint32)
    key = jnp.where(bits >= _sign(), ~bits, bits | _sign())
    out_ref[0] = key


def _ce_keys(logits, target_long, off):
    out = pl.pallas_call(
        _ce_key_body,
        grid=(BH, 512 // RB),
        in_specs=[
            pl.BlockSpec((1, C, RB, 512), lambda b, j: (b + off, 0, j, 0)),
            pl.BlockSpec((1, RB, 512), lambda b, j: (b + off, j, 0)),
        ],
        out_specs=pl.BlockSpec((1, RB, 512), lambda b, j: (b, j, 0)),
        out_shape=jax.ShapeDtypeStruct((BH, 512, 512), jnp.uint32),
    )(logits, target_long)
    return out.reshape(BH * N)


def _make_hist_kernel(shift, filtered):
    mesh = plsc.VectorSubcoreMesh(core_axis_name="c", subcore_axis_name="s")

    @functools.partial(
        pl.kernel,
        out_type=(
            jax.ShapeDtypeStruct((BH, 16), jnp.int32),
            jax.ShapeDtypeStruct((BH, 16), jnp.int32),
            jax.ShapeDtypeStruct((BH, 16), jnp.float32),
        ),
        mesh=mesh,
        compiler_params=pltpu.CompilerParams(needs_layout_passes=False),
        scratch_types=[
            pltpu.VMEM((NB * 16,), jnp.int32),
            pltpu.VMEM((NB * 16,), jnp.float32),
            pltpu.VMEM((NB,), jnp.int32),
            pltpu.VMEM((NB,), jnp.float32),
            pltpu.VMEM((NB,), jnp.int32),
            pltpu.VMEM((NB,), jnp.float32),
            pltpu.VMEM((CHUNK,), jnp.uint32),
            pltpu.VMEM((CHUNK,), jnp.uint32),
            pltpu.VMEM((32,), jnp.int32),
            pltpu.VMEM((16,), jnp.int32),
            pltpu.VMEM((16,), jnp.int32),
            pltpu.VMEM((16,), jnp.float32),
            pltpu.VMEM_SHARED((16, NB), jnp.int32),
            pltpu.VMEM_SHARED((16, NB), jnp.float32),
            pltpu.SemaphoreType.DMA,
            pltpu.SemaphoreType.DMA,
        ],
    )
    def hist(keys_hbm, filt_hbm, p_out, ca_out, sa_out,
             cnt_v, sum_v, cnt_f, sum_f, tmp_c, tmp_s, buf0, buf1, filt_v,
             ob_p, ob_c, ob_s, shared_c, shared_s, sem0, sem1):
        core = lax.axis_index("c")
        sidx = lax.axis_index("s")
        sample = core * (BH // 2) + sidx // SH
        shard = sidx % SH
        base = sample * N + shard * PER_W

        zi = jnp.zeros((16,), jnp.int32)
        zf = jnp.zeros((16,), jnp.float32)

        def zbody(i, carry):
            for u in range(4):
                off = (i * 4 + u) * 16
                cnt_v[pl.ds(off, 16)] = zi
                sum_v[pl.ds(off, 16)] = zf
            return carry

        lax.fori_loop(0, (NB * 16) // 64, zbody, 0)

        pltpu.sync_copy(filt_hbm, filt_v)
        if filtered:
            pfx = plsc.load_gather(filt_v, [jnp.full((16,), sample, jnp.int32)])
            pfx = pfx.astype(jnp.uint32)
            krem = plsc.load_gather(
                filt_v, [jnp.full((16,), 16 + sample, jnp.int32)])
        else:
            krem = jnp.full((16,), K, jnp.int32)

        lane = lax.broadcasted_iota(jnp.int32, (16,), 0)
        ones = jnp.ones((16,), jnp.int32)

        def make_inner(buf):
            def inner(g, carry):
                kvs = [buf[pl.ds((g * UNROLL + u) * 16, 16)]
                       for u in range(UNROLL)]
                idxs = []
                vals = []
                msks = []
                for kv in kvs:
                    bin_ = ((kv >> shift) & jnp.uint32(NB - 1)).astype(jnp.int32)
                    idxs.append((bin_ << 4) | lane)
                    bits = jnp.where(kv >= _sign(), kv & _mant(), ~kv)
                    vals.append(lax.bitcast_convert_type(bits, jnp.float32))
                    if filtered:
                        msks.append((kv >> SHIFT1) == pfx)
                for u in range(UNROLL):
                    if filtered:
                        plsc.addupdate_scatter(cnt_v, [idxs[u]], ones,
                                               mask=msks[u])
                        plsc.addupdate_scatter(sum_v, [idxs[u]], vals[u],
                                               mask=msks[u])
                    else:
                        plsc.addupdate_scatter(cnt_v, [idxs[u]], ones)
                        plsc.addupdate_scatter(sum_v, [idxs[u]], vals[u])
                return carry
            return inner

        bufs = (buf0, buf1)
        sems = (sem0, sem1)
        copies = [None, None]
        copies[0] = pltpu.async_copy(
            keys_hbm.at[pl.ds(base, CHUNK)], buf0, sem0)
        for ci in range(N_CHUNKS):
            cur = ci % 2
            nxt = (ci + 1) % 2
            if ci + 1 < N_CHUNKS:
                copies[nxt] = pltpu.async_copy(
                    keys_hbm.at[pl.ds(base + (ci + 1) * CHUNK, CHUNK)],
                    bufs[nxt], sems[nxt])
            copies[cur].wait()
            lax.fori_loop(0, CHUNK // (16 * UNROLL), make_inner(bufs[cur]), 0)

        def fold(g, carry):
            base16 = g * 256
            gidx = base16 + lane * 16
            ci_acc = plsc.load_gather(cnt_v, [gidx])
            cf_acc = plsc.load_gather(sum_v, [gidx])
            for l in range(1, 16):
                ci_acc = ci_acc + plsc.load_gather(cnt_v, [gidx + l])
                cf_acc = cf_acc + plsc.load_gather(sum_v, [gidx + l])
            cnt_f[pl.ds(g * 16, 16)] = ci_acc
            sum_f[pl.ds(g * 16, 16)] = cf_acc
            return carry

        lax.fori_loop(0, NB // 16, fold, 0)

        pltpu.sync_copy(cnt_f, shared_c.at[sidx])
        pltpu.sync_copy(sum_f, shared_s.at[sidx])
        plsc.subcore_barrier()

        @pl.when(shard == 0)
        def _select():
            for q in range(1, SH):
                pltpu.sync_copy(shared_c.at[sidx + q], tmp_c)
                pltpu.sync_copy(shared_s.at[sidx + q], tmp_s)

                def mbody(g, carry):
                    off = g * 16
                    cnt_f[pl.ds(off, 16)] = (cnt_f[pl.ds(off, 16)]
                                             + tmp_c[pl.ds(off, 16)])
                    sum_f[pl.ds(off, 16)] = (sum_f[pl.ds(off, 16)]
                                             + tmp_s[pl.ds(off, 16)])
                    return carry

                lax.fori_loop(0, NB // 16, mbody, 0)

            def tbody(g, carry):
                ai, af = carry
                return (ai + cnt_f[pl.ds(g * 16, 16)],
                        af + sum_f[pl.ds(g * 16, 16)])

            ti, tf = lax.fori_loop(
                0, NB // 16, tbody,
                (jnp.zeros((16,), jnp.int32), jnp.zeros((16,), jnp.float32)))
            tot_c = jnp.sum(ti)
            tot_s = jnp.sum(tf)

            def sbody(g, carry):
                run_c, run_s, accp, accc, accs = carry
                v = cnt_f[pl.ds(g * 16, 16)]
                w = sum_f[pl.ds(g * 16, 16)]
                incl_c = run_c + plsc.cumsum(v)
                incl_s = run_s + plsc.cumsum(w)
                gt_c = tot_c - incl_c
                gt_s = tot_s - incl_s
                flag = (gt_c < krem) & ((gt_c + v) >= krem)
                accp = accp + jnp.where(flag, g * 16 + lane, 0)
                accc = accc + jnp.where(flag, gt_c, 0)
                accs = accs + jnp.where(flag, gt_s, jnp.float32(0.0))
                return (run_c + jnp.sum(v), run_s + jnp.sum(w),
                        accp, accc, accs)

            z16i = jnp.zeros((16,), jnp.int32)
            _, _, accp, accc, accs = lax.fori_loop(
                0, NB // 16, sbody,
                (jnp.int32(0), jnp.float32(0.0), z16i, z16i,
                 jnp.zeros((16,), jnp.float32)))
            ob_p[...] = accp
            ob_c[...] = accc
            ob_s[...] = accs
            pltpu.sync_copy(ob_p, p_out.at[sample])
            pltpu.sync_copy(ob_c, ca_out.at[sample])
            pltpu.sync_copy(ob_s, sa_out.at[sample])

    return hist


@functools.lru_cache(maxsize=None)
def _hist_kernels():
    return (_make_hist_kernel(SHIFT1, filtered=False),
            _make_hist_kernel(SHIFT2, filtered=True))


def _decode_key(key_u32):
    neg = key_u32 < _sign()
    bits = jnp.where(neg, ~key_u32, key_u32 & _mant())
    return lax.bitcast_convert_type(bits, jnp.float32)


def _half(logits, target_long, off):
    _hist_pass1, _hist_pass2 = _hist_kernels()
    keys = _ce_keys(logits, target_long, off)

    p1v, ca1v, sa1v = _hist_pass1(keys, jnp.zeros((32,), jnp.int32))
    P1 = p1v.sum(axis=1)
    cA1 = ca1v.sum(axis=1)
    sA1 = sa1v.sum(axis=1)
    r1 = jnp.full((BH,), K, jnp.int32) - cA1

    filt = jnp.concatenate([
        jnp.zeros((16,), jnp.int32).at[:BH].set(P1),
        jnp.zeros((16,), jnp.int32).at[:BH].set(r1),
    ])
    p2v, ca2v, sa2v = _hist_pass2(keys, filt)
    P2 = p2v.sum(axis=1)
    cA2 = ca2v.sum(axis=1)
    sA2 = sa2v.sum(axis=1)

    r = (r1 - cA2).astype(jnp.float32)
    key_mid = ((P1.astype(jnp.uint32) << SHIFT1)
               | (P2.astype(jnp.uint32) << SHIFT2)
               | jnp.uint32(1 << (SHIFT2 - 1)))
    t_mid = _decode_key(key_mid)
    return (sA1 + sA2 + r * t_mid) / jnp.float32(K)


def kernel(logits, target_long):
    return jnp.mean(_half(logits, target_long, 0))


# speedup vs baseline: 1.2558x; 1.0536x over previous
import functools

import jax
import jax.numpy as jnp
from jax import lax
from jax.experimental import pallas as pl
from jax.experimental.pallas import tpu as pltpu
from jax.experimental.pallas import tpu_sc as plsc

B = 8
C = 19
N = 512 * 512
K = max(int(N * 0.2), 1)

NB = 512
SHIFT1 = 23
SHIFT2 = 14

NW = 32
BH = 8
SH = NW // BH
PER_W = (BH * N) // NW
CHUNK = 8192
N_CHUNKS = PER_W // CHUNK
UNROLL = 8

RB = 256

def _sign():
    return jnp.uint32(0x80000000)


def _mant():
    return jnp.uint32(0x7FFFFFFF)


def _ce_key_body(logits_ref, tgt_ref, out_ref):
    x = logits_ref[0]
    t = tgt_ref[0]
    cls = lax.broadcasted_iota(jnp.int32, (C, RB, 512), 0)
    xt = jnp.sum(jnp.where(cls == t[None], x, 0.0), axis=0)
    m = jnp.max(x, axis=0)
    s = jnp.sum(jnp.exp(x - m[None]), axis=0)
    ce = jnp.log(s) + m - xt
    bits = lax.bitcast_convert_type(ce, jnp.uint32)
    key = jnp.where(bits >= _sign(), ~bits, bits | _sign())
    out_ref[0] = key


def _ce_keys(logits, target_long, off):
    out = pl.pallas_call(
        _ce_key_body,
        grid=(BH, 512 // RB),
        in_specs=[
            pl.BlockSpec((1, C, RB, 512), lambda b, j: (b + off, 0, j, 0)),
            pl.BlockSpec((1, RB, 512), lambda b, j: (b + off, j, 0)),
        ],
        out_specs=pl.BlockSpec((1, RB, 512), lambda b, j: (b, j, 0)),
        out_shape=jax.ShapeDtypeStruct((BH, 512, 512), jnp.uint32),
    )(logits, target_long)
    return out.reshape(BH * N)


def _make_hist_kernel(shift, filtered):
    mesh = plsc.VectorSubcoreMesh(core_axis_name="c", subcore_axis_name="s")

    @functools.partial(
        pl.kernel,
        out_type=(
            jax.ShapeDtypeStruct((BH, 16), jnp.int32),
            jax.ShapeDtypeStruct((BH, 16), jnp.int32),
            jax.ShapeDtypeStruct((BH, 16), jnp.float32),
        ),
        mesh=mesh,
        compiler_params=pltpu.CompilerParams(needs_layout_passes=False),
        scratch_types=[
            pltpu.VMEM((NB * 16,), jnp.int32),
            pltpu.VMEM((NB * 16,), jnp.float32),
            pltpu.VMEM((NB,), jnp.int32),
            pltpu.VMEM((NB,), jnp.float32),
            pltpu.VMEM((NB,), jnp.int32),
            pltpu.VMEM((NB,), jnp.float32),
            pltpu.VMEM((CHUNK,), jnp.uint32),
            pltpu.VMEM((CHUNK,), jnp.uint32),
            pltpu.VMEM((32,), jnp.int32),
            pltpu.VMEM((16,), jnp.int32),
            pltpu.VMEM((16,), jnp.int32),
            pltpu.VMEM((16,), jnp.float32),
            pltpu.VMEM_SHARED((16, NB), jnp.int32),
            pltpu.VMEM_SHARED((16, NB), jnp.float32),
            pltpu.SemaphoreType.DMA,
            pltpu.SemaphoreType.DMA,
        ],
    )
    def hist(keys_hbm, filt_hbm, p_out, ca_out, sa_out,
             cnt_v, sum_v, cnt_f, sum_f, tmp_c, tmp_s, buf0, buf1, filt_v,
             ob_p, ob_c, ob_s, shared_c, shared_s, sem0, sem1):
        core = lax.axis_index("c")
        sidx = lax.axis_index("s")
        sample = core * (BH // 2) + sidx // SH
        shard = sidx % SH
        base = sample * N + shard * PER_W

        zi = jnp.zeros((16,), jnp.int32)
        zf = jnp.zeros((16,), jnp.float32)

        def zbody(i, carry):
            for u in range(4):
                off = (i * 4 + u) * 16
                cnt_v[pl.ds(off, 16)] = zi
                sum_v[pl.ds(off, 16)] = zf
            return carry

        lax.fori_loop(0, (NB * 16) // 64, zbody, 0)

        pltpu.sync_copy(filt_hbm, filt_v)
        if filtered:
            pfx = plsc.load_gather(filt_v, [jnp.full((16,), sample, jnp.int32)])
            pfx = pfx.astype(jnp.uint32)
            krem = plsc.load_gather(
                filt_v, [jnp.full((16,), 16 + sample, jnp.int32)])
        else:
            krem = jnp.full((16,), K, jnp.int32)

        lane = lax.broadcasted_iota(jnp.int32, (16,), 0)
        ones = jnp.ones((16,), jnp.int32)

        def make_inner(buf):
            def inner(g, carry):
                kvs = [buf[pl.ds((g * UNROLL + u) * 16, 16)]
                       for u in range(UNROLL)]
                idxs = []
                vals = []
                msks = []
                for kv in kvs:
                    bin_ = ((kv >> shift) & jnp.uint32(NB - 1)).astype(jnp.int32)
                    idxs.append((bin_ << 4) | lane)
                    bits = jnp.where(kv >= _sign(), kv & _mant(), ~kv)
                    vals.append(lax.bitcast_convert_type(bits, jnp.float32))
                    if filtered:
                        msks.append((kv >> SHIFT1) == pfx)
                for u in range(UNROLL):
                    if filtered:
                        plsc.addupdate_scatter(cnt_v, [idxs[u]], ones,
                                               mask=msks[u])
                        plsc.addupdate_scatter(sum_v, [idxs[u]], vals[u],
                                               mask=msks[u])
                    else:
                        plsc.addupdate_scatter(cnt_v, [idxs[u]], ones)
                        plsc.addupdate_scatter(sum_v, [idxs[u]], vals[u])
                return carry
            return inner

        bufs = (buf0, buf1)
        sems = (sem0, sem1)
        copies = [None, None]
        copies[0] = pltpu.async_copy(
            keys_hbm.at[pl.ds(base, CHUNK)], buf0, sem0)
        for ci in range(N_CHUNKS):
            cur = ci % 2
            nxt = (ci + 1) % 2
            if ci + 1 < N_CHUNKS:
                copies[nxt] = pltpu.async_copy(
                    keys_hbm.at[pl.ds(base + (ci + 1) * CHUNK, CHUNK)],
                    bufs[nxt], sems[nxt])
            copies[cur].wait()
            lax.fori_loop(0, CHUNK // (16 * UNROLL), make_inner(bufs[cur]), 0)

        def fold(g, carry):
            base16 = g * 256
            gidx = base16 + lane * 16
            ci_acc = plsc.load_gather(cnt_v, [gidx])
            cf_acc = plsc.load_gather(sum_v, [gidx])
            for l in range(1, 16):
                ci_acc = ci_acc + plsc.load_gather(cnt_v, [gidx + l])
                cf_acc = cf_acc + plsc.load_gather(sum_v, [gidx + l])
            cnt_f[pl.ds(g * 16, 16)] = ci_acc
            sum_f[pl.ds(g * 16, 16)] = cf_acc
            return carry

        lax.fori_loop(0, NB // 16, fold, 0)

        pltpu.sync_copy(cnt_f, shared_c.at[sidx])
        pltpu.sync_copy(sum_f, shared_s.at[sidx])
        plsc.subcore_barrier()

        @pl.when(shard == 0)
        def _select():
            for q in range(1, SH):
                pltpu.sync_copy(shared_c.at[sidx + q], tmp_c)
                pltpu.sync_copy(shared_s.at[sidx + q], tmp_s)

                def mbody(g, carry):
                    off = g * 16
                    cnt_f[pl.ds(off, 16)] = (cnt_f[pl.ds(off, 16)]
                                             + tmp_c[pl.ds(off, 16)])
                    sum_f[pl.ds(off, 16)] = (sum_f[pl.ds(off, 16)]
                                             + tmp_s[pl.ds(off, 16)])
                    return carry

                lax.fori_loop(0, NB // 16, mbody, 0)

            def tbody(g, carry):
                ai, af = carry
                return (ai + cnt_f[pl.ds(g * 16, 16)],
                        af + sum_f[pl.ds(g * 16, 16)])

            ti, tf = lax.fori_loop(
                0, NB // 16, tbody,
                (jnp.zeros((16,), jnp.int32), jnp.zeros((16,), jnp.float32)))
            tot_c = jnp.sum(ti)
            tot_s = jnp.sum(tf)

            def sbody(g, carry):
                run_c, run_s, accp, accc, accs = carry
                v = cnt_f[pl.ds(g * 16, 16)]
                w = sum_f[pl.ds(g * 16, 16)]
                incl_c = run_c + plsc.cumsum(v)
                incl_s = run_s + plsc.cumsum(w)
                gt_c = tot_c - incl_c
                gt_s = tot_s - incl_s
                flag = (gt_c < krem) & ((gt_c + v) >= krem)
                accp = accp + jnp.where(flag, g * 16 + lane, 0)
                accc = accc + jnp.where(flag, gt_c, 0)
                accs = accs + jnp.where(flag, gt_s, jnp.float32(0.0))
                return (run_c + jnp.sum(v), run_s + jnp.sum(w),
                        accp, accc, accs)

            z16i = jnp.zeros((16,), jnp.int32)
            _, _, accp, accc, accs = lax.fori_loop(
                0, NB // 16, sbody,
                (jnp.int32(0), jnp.float32(0.0), z16i, z16i,
                 jnp.zeros((16,), jnp.float32)))
            ob_p[...] = accp
            ob_c[...] = accc
            ob_s[...] = accs
            pltpu.sync_copy(ob_p, p_out.at[sample])
            pltpu.sync_copy(ob_c, ca_out.at[sample])
            pltpu.sync_copy(ob_s, sa_out.at[sample])

    return hist


@functools.lru_cache(maxsize=None)
def _hist_kernels():
    return (_make_hist_kernel(SHIFT1, filtered=False),
            _make_hist_kernel(SHIFT2, filtered=True))


def _decode_key(key_u32):
    neg = key_u32 < _sign()
    bits = jnp.where(neg, ~key_u32, key_u32 & _mant())
    return lax.bitcast_convert_type(bits, jnp.float32)


def _half(logits, target_long, off):
    _hist_pass1, _hist_pass2 = _hist_kernels()
    keys = _ce_keys(logits, target_long, off)

    p1v, ca1v, sa1v = _hist_pass1(keys, jnp.zeros((32,), jnp.int32))
    P1 = p1v.sum(axis=1)
    cA1 = ca1v.sum(axis=1)
    sA1 = sa1v.sum(axis=1)
    r1 = jnp.full((BH,), K, jnp.int32) - cA1

    filt = jnp.concatenate([
        jnp.zeros((16,), jnp.int32).at[:BH].set(P1),
        jnp.zeros((16,), jnp.int32).at[:BH].set(r1),
    ])
    p2v, ca2v, sa2v = _hist_pass2(keys, filt)
    P2 = p2v.sum(axis=1)
    cA2 = ca2v.sum(axis=1)
    sA2 = sa2v.sum(axis=1)

    r = (r1 - cA2).astype(jnp.float32)
    key_mid = ((P1.astype(jnp.uint32) << SHIFT1)
               | (P2.astype(jnp.uint32) << SHIFT2)
               | jnp.uint32(1 << (SHIFT2 - 1)))
    t_mid = _decode_key(key_mid)
    return (sA1 + sA2 + r * t_mid) / jnp.float32(K)


def kernel(logits, target_long):
    return jnp.mean(_half(logits, target_long, 0))
